# baseline (speedup 1.0000x reference)
import jax
import jax.numpy as jnp
from jax.experimental import pallas as pl
from jax.experimental.pallas import tpu as pltpu

K = 1024
D = 64
CC = 0.25
NRQ = 3
N_TOTAL = 16 * 1024
BN = 2048
N_STEPS = N_TOTAL // BN
NSPLIT = 4


def _rowsum64(a):
    acc = a
    for i in range(1, 8):
        acc = acc + pltpu.roll(a, 64 - 8 * i, 1)
    t = acc[:, 0:4] + acc[:, 4:8]
    u = t[:, 0:2] + t[:, 2:4]
    return u[:, 0:1] + u[:, 1:2]


def _colsum64(a):
    acc = a[0:8, :]
    for i in range(1, 8):
        acc = acc + a[8 * i:8 * i + 8, :]
    t = acc[0:4, :] + acc[4:8, :]
    u = t[0:2, :] + t[2:4, :]
    return u[0:1, :] + u[1:2, :]


def _rvq_kernel(x_ref, w0_ref, w1_ref, w2_ref, w3_ref,
                wt0_ref, wt1_ref, wt2_ref, wt3_ref,
                loss_ref, quant_ref, perp_ref, enc_ref,
                cn_scr, hist_scr, sse_scr):
    i = pl.program_id(0)

    @pl.when(i == 0)
    def _init():
        hist_scr[...] = jnp.zeros_like(hist_scr)
        sse_scr[...] = jnp.zeros_like(sse_scr)
        for s, wt_ref in enumerate((wt0_ref, wt1_ref, wt2_ref, wt3_ref)):
            wt = wt_ref[...]
            cn_scr[s:s + 1, :] = _colsum64(wt * wt)

    x = x_ref[...]
    H = BN // NSPLIT
    sse = sse_scr[...]
    hist = hist_scr[...]
    iota_k = jax.lax.broadcasted_iota(jnp.int32, (H, K), 1)

    xs = [x[h * H:(h + 1) * H, :] for h in range(NSPLIT)]
    quants = [jnp.zeros_like(xs[h]) for h in range(NSPLIT)]
    last_onehots = [None] * NSPLIT
    for s, w_ref in enumerate((w0_ref, w1_ref, w2_ref, w3_ref)):
        w = w_ref[...]
        cn = cn_scr[s:s + 1, :]
        for h in range(NSPLIT):
            res = xs[h] - quants[h]
            rn = _rowsum64(res * res)
            xw = jax.lax.dot_general(
                res, w, (((1,), (1,)), ((), ())),
                preferred_element_type=jnp.float32)
            dist = rn + cn - 2.0 * xw
            dmin = jnp.min(dist, axis=1, keepdims=True)
            cand = jnp.where(dist == dmin, iota_k, K)
            idx = jnp.min(cand, axis=1, keepdims=True)
            onehot = (iota_k == idx).astype(jnp.float32)
            q = jax.lax.dot_general(
                onehot, w, (((1,), (0,)), ((), ())),
                preferred_element_type=jnp.float32)
            err = q - res
            sse = sse + jnp.sum(err * err).reshape(1, 1)
            hist = hist + jnp.sum(onehot, axis=0, keepdims=True)
            quants[h] = quants[h] + q
            last_onehots[h] = onehot

    sse_scr[...] = sse
    hist_scr[...] = hist
    for h in range(NSPLIT):
        quant_ref[h * H:(h + 1) * H, :] = xs[h] + (quants[h] - xs[h])
        enc_ref[h * H:(h + 1) * H, :] = last_onehots[h]

    @pl.when(i == N_STEPS - 1)
    def _finish():
        total = sse_scr[0, 0]
        loss_ref[...] = ((1.0 + CC) * total / (N_TOTAL * D)).reshape(1, 1)
        p = hist_scr[...] / (N_TOTAL * (NRQ + 1))
        ent = -jnp.sum(p * jnp.log(p + 1e-10))
        perp_ref[...] = jnp.exp(ent).reshape(1, 1)


@jax.jit
def kernel(inputs, emb_w, res_w0, res_w1, res_w2):
    input_shape = inputs.shape
    flat = inputs.reshape(-1, D)

    out_shapes = (
        jax.ShapeDtypeStruct((1, 1), jnp.float32),
        jax.ShapeDtypeStruct((N_TOTAL, D), jnp.float32),
        jax.ShapeDtypeStruct((1, 1), jnp.float32),
        jax.ShapeDtypeStruct((N_TOTAL, K), jnp.float32),
    )
    w_spec = pl.BlockSpec((K, D), lambda i: (0, 0))
    wt_spec = pl.BlockSpec((D, K), lambda i: (0, 0))
    loss, quant, perp, enc = pl.pallas_call(
        _rvq_kernel,
        grid=(N_STEPS,),
        in_specs=[
            pl.BlockSpec((BN, D), lambda i: (i, 0)),
            w_spec, w_spec, w_spec, w_spec,
            wt_spec, wt_spec, wt_spec, wt_spec,
        ],
        out_specs=(
            pl.BlockSpec((1, 1), lambda i: (0, 0)),
            pl.BlockSpec((BN, D), lambda i: (i, 0)),
            pl.BlockSpec((1, 1), lambda i: (0, 0)),
            pl.BlockSpec((BN, K), lambda i: (i, 0)),
        ),
        out_shape=out_shapes,
        scratch_shapes=[
            pltpu.VMEM((4, K), jnp.float32),
            pltpu.VMEM((1, K), jnp.float32),
            pltpu.VMEM((1, 1), jnp.float32),
        ],
    )(flat, emb_w, res_w0, res_w1, res_w2,
      emb_w.T, res_w0.T, res_w1.T, res_w2.T)

    return (loss[0, 0], quant.reshape(input_shape), perp[0, 0], enc)


# speedup vs baseline: 1.0168x; 1.0168x over previous
import jax
import jax.numpy as jnp
from jax.experimental import pallas as pl
from jax.experimental.pallas import tpu as pltpu

K = 1024
D = 64
CC = 0.25
NRQ = 3
N_TOTAL = 16 * 1024
BN = 2048
N_STEPS = N_TOTAL // BN
NSPLIT = 2


def _rowsum64(a):
    acc = a
    for i in range(1, 8):
        acc = acc + pltpu.roll(a, 64 - 8 * i, 1)
    t = acc[:, 0:4] + acc[:, 4:8]
    u = t[:, 0:2] + t[:, 2:4]
    return u[:, 0:1] + u[:, 1:2]


def _colsum64(a):
    acc = a[0:8, :]
    for i in range(1, 8):
        acc = acc + a[8 * i:8 * i + 8, :]
    t = acc[0:4, :] + acc[4:8, :]
    u = t[0:2, :] + t[2:4, :]
    return u[0:1, :] + u[1:2, :]


def _rvq_kernel(x_ref, w0_ref, w1_ref, w2_ref, w3_ref,
                wt0_ref, wt1_ref, wt2_ref, wt3_ref,
                loss_ref, quant_ref, perp_ref, enc_ref,
                cn_scr, hist_scr, sse_scr):
    i = pl.program_id(0)

    @pl.when(i == 0)
    def _init():
        hist_scr[...] = jnp.zeros_like(hist_scr)
        sse_scr[...] = jnp.zeros_like(sse_scr)
        for s, wt_ref in enumerate((wt0_ref, wt1_ref, wt2_ref, wt3_ref)):
            wt = wt_ref[...]
            cn_scr[s:s + 1, :] = _colsum64(wt * wt)

    x = x_ref[...]
    H = BN // NSPLIT
    sse = sse_scr[...]
    hist = hist_scr[...]
    iota_k = jax.lax.broadcasted_iota(jnp.int32, (H, K), 1)

    xs = [x[h * H:(h + 1) * H, :] for h in range(NSPLIT)]
    quants = [jnp.zeros_like(xs[h]) for h in range(NSPLIT)]
    last_onehots = [None] * NSPLIT
    for s, w_ref in enumerate((w0_ref, w1_ref, w2_ref, w3_ref)):
        w = w_ref[...]
        cn = cn_scr[s:s + 1, :]
        for h in range(NSPLIT):
            res = xs[h] - quants[h]
            rn = _rowsum64(res * res)
            xw = jax.lax.dot_general(
                res, w, (((1,), (1,)), ((), ())),
                preferred_element_type=jnp.float32)
            dist = rn + cn - 2.0 * xw
            dmin = jnp.min(dist, axis=1, keepdims=True)
            cand = jnp.where(dist == dmin, iota_k, K)
            idx = jnp.min(cand, axis=1, keepdims=True)
            onehot = (iota_k == idx).astype(jnp.float32)
            q = jax.lax.dot_general(
                onehot, w, (((1,), (0,)), ((), ())),
                preferred_element_type=jnp.float32)
            err = q - res
            sse = sse + jnp.sum(err * err).reshape(1, 1)
            hist = hist + jnp.sum(onehot, axis=0, keepdims=True)
            quants[h] = quants[h] + q
            last_onehots[h] = onehot

    sse_scr[...] = sse
    hist_scr[...] = hist
    for h in range(NSPLIT):
        quant_ref[h * H:(h + 1) * H, :] = xs[h] + (quants[h] - xs[h])
        enc_ref[h * H:(h + 1) * H, :] = last_onehots[h]

    @pl.when(i == N_STEPS - 1)
    def _finish():
        total = sse_scr[0, 0]
        loss_ref[...] = ((1.0 + CC) * total / (N_TOTAL * D)).reshape(1, 1)
        p = hist_scr[...] / (N_TOTAL * (NRQ + 1))
        ent = -jnp.sum(p * jnp.log(p + 1e-10))
        perp_ref[...] = jnp.exp(ent).reshape(1, 1)


@jax.jit
def kernel(inputs, emb_w, res_w0, res_w1, res_w2):
    input_shape = inputs.shape
    flat = inputs.reshape(-1, D)

    out_shapes = (
        jax.ShapeDtypeStruct((1, 1), jnp.float32),
        jax.ShapeDtypeStruct((N_TOTAL, D), jnp.float32),
        jax.ShapeDtypeStruct((1, 1), jnp.float32),
        jax.ShapeDtypeStruct((N_TOTAL, K), jnp.float32),
    )
    w_spec = pl.BlockSpec((K, D), lambda i: (0, 0))
    wt_spec = pl.BlockSpec((D, K), lambda i: (0, 0))
    loss, quant, perp, enc = pl.pallas_call(
        _rvq_kernel,
        grid=(N_STEPS,),
        in_specs=[
            pl.BlockSpec((BN, D), lambda i: (i, 0)),
            w_spec, w_spec, w_spec, w_spec,
            wt_spec, wt_spec, wt_spec, wt_spec,
        ],
        out_specs=(
            pl.BlockSpec((1, 1), lambda i: (0, 0)),
            pl.BlockSpec((BN, D), lambda i: (i, 0)),
            pl.BlockSpec((1, 1), lambda i: (0, 0)),
            pl.BlockSpec((BN, K), lambda i: (i, 0)),
        ),
        out_shape=out_shapes,
        scratch_shapes=[
            pltpu.VMEM((4, K), jnp.float32),
            pltpu.VMEM((1, K), jnp.float32),
            pltpu.VMEM((1, 1), jnp.float32),
        ],
    )(flat, emb_w, res_w0, res_w1, res_w2,
      emb_w.T, res_w0.T, res_w1.T, res_w2.T)

    return (loss[0, 0], quant.reshape(input_shape), perp[0, 0], enc)
